# baseline (speedup 1.0000x reference)
import functools

import jax
import jax.numpy as jnp
from jax import lax
from jax.experimental import pallas as pl
from jax.experimental.pallas import tpu as pltpu
from jax.experimental.pallas import tpu_sc as plsc

H = 128
HM = 153
HMP = 160
N = 10000
E = 160000
F32 = jnp.float32

NCORES = 2
NSUB = 16
NW = NCORES * NSUB
CHUNK = 128
NJ = 40
E2 = NW * NJ * CHUNK
NCH = E2 // CHUNK
NB = 2
NG = NJ // NB
NP = 10240
RPT = NP // NSUB
BR = 1024


def _inv_sqrt(d):
    safe = jnp.where(d > 0, d, 1.0)
    return jnp.where(d > 0, lax.rsqrt(safe), 0.0)


def _mlp_pair_body(rank1, x_ref, deg_ref, W1_ref, b1_ref, W2_ref, b2_ref,
                   W3_ref, b3_ref, oa_ref, ob_ref):
    dsi = _inv_sqrt(deg_ref[...])
    x = x_ref[...]
    for k, out in ((0, oa_ref), (1, ob_ref)):
        if rank1:
            h = x * W1_ref[k] + b1_ref[k]
        else:
            h = jnp.dot(x, W1_ref[k], preferred_element_type=F32) + b1_ref[k]
        h = jnp.maximum(h, 0.0)
        h = jnp.maximum(jnp.dot(h, W2_ref[k], preferred_element_type=F32) + b2_ref[k], 0.0)
        h = jnp.maximum(jnp.dot(h, W3_ref[k], preferred_element_type=F32) + b3_ref[k], 0.0)
        out[...] = h * dsi


def _mlp_pair(x, deg, W1, b1, W2, b2, W3, b3, rank1):
    grid = (NP // BR,)
    kdim = 1 if rank1 else H
    full = lambda *s: pl.BlockSpec(s, lambda i: (0,) * len(s))
    return pl.pallas_call(
        functools.partial(_mlp_pair_body, rank1),
        grid=grid,
        in_specs=[
            pl.BlockSpec((BR, kdim), lambda i: (i, 0)),
            pl.BlockSpec((BR, 1), lambda i: (i, 0)),
            full(2, kdim, HMP), full(2, 1, HMP),
            full(2, HMP, HMP), full(2, 1, HMP),
            full(2, HMP, H), full(2, 1, H),
        ],
        out_specs=[pl.BlockSpec((BR, H), lambda i: (i, 0))] * 2,
        out_shape=[jax.ShapeDtypeStruct((NP, H), F32)] * 2,
    )(x, deg, W1, b1, W2, b2, W3, b3)


def _combine_body(rank1, pp_ref, pn_ref, deg_ref, prev_ref, Wa_ref, Wb_ref,
                  Wc_ref, b_ref, out_ref):
    dti = _inv_sqrt(deg_ref[...])
    a = (pp_ref[0] + pp_ref[1]) * dti
    c = (pn_ref[0] + pn_ref[1]) * dti
    acc = jnp.dot(a, Wa_ref[...], preferred_element_type=F32)
    acc += jnp.dot(c, Wb_ref[...], preferred_element_type=F32)
    if rank1:
        acc += prev_ref[...] * Wc_ref[...]
    else:
        acc += jnp.dot(prev_ref[...], Wc_ref[...], preferred_element_type=F32)
    out_ref[...] = acc + b_ref[...]


def _combine(pp, pn, deg, prev, Wa, Wb, Wc, b, rank1):
    grid = (NP // BR,)
    kdim = 1 if rank1 else H
    full = lambda *s: pl.BlockSpec(s, lambda i: (0,) * len(s))
    return pl.pallas_call(
        functools.partial(_combine_body, rank1),
        grid=grid,
        in_specs=[
            pl.BlockSpec((2, BR, H), lambda i: (0, i, 0)),
            pl.BlockSpec((2, BR, H), lambda i: (0, i, 0)),
            pl.BlockSpec((BR, 1), lambda i: (i, 0)),
            pl.BlockSpec((BR, kdim), lambda i: (i, 0)),
            full(H, H), full(H, H), full(kdim, H), full(1, H),
        ],
        out_specs=pl.BlockSpec((BR, H), lambda i: (i, 0)),
        out_shape=jax.ShapeDtypeStruct((NP, H), F32),
    )(pp, pn, deg, prev, Wa, Wb, Wc, b)


def _conv2_body(z0, s0, t0, z1, s1, t1, zrows, out0, out1,
                acc, sbuf, tbuf, rows, semg, sems):
    cid = lax.axis_index("c")
    sid = lax.axis_index("s")
    wid = sid * NCORES + cid
    base = sid * RPT
    for (z, s, t, out) in ((z0, s0, t0, out0), (z1, s1, t1, out1)):
        pltpu.sync_copy(zrows, acc.at[pl.ds(base, RPT)])
        plsc.subcore_barrier()

        @pl.loop(0, NJ)
        def _(j):
            off = (wid + NW * j) * CHUNK
            pltpu.sync_copy(s.at[pl.ds(off, CHUNK)], sbuf)
            pltpu.sync_copy(t.at[pl.ds(off, CHUNK)], tbuf)
            pltpu.async_copy(z.at[sbuf], rows.at[0], semg[0]).wait()
            pltpu.sync_copy(rows.at[0], acc.at[tbuf], add=True)

        plsc.subcore_barrier()
        pltpu.sync_copy(acc.at[pl.ds(base, RPT)],
                        out.at[pl.ds(cid * NP + base, RPT)])


@functools.cache
def _conv2_kernel():
    mesh = plsc.VectorSubcoreMesh(core_axis_name="c", subcore_axis_name="s")
    return pl.kernel(
        _conv2_body,
        mesh=mesh,
        out_type=[jax.ShapeDtypeStruct((NCORES * NP, H), F32)] * 2,
        scratch_types=[
            pltpu.VMEM_SHARED((NP, H), F32),
            pltpu.VMEM((CHUNK,), jnp.int32),
            pltpu.VMEM((CHUNK,), jnp.int32),
            pltpu.VMEM((NB, CHUNK, H), F32),
            [pltpu.SemaphoreType.DMA] * NB,
            [pltpu.SemaphoreType.DMA] * NB,
        ],
    )


def _conv_pair(z0, ei0, z1, ei1, zrows):
    p0, p1 = _conv2_kernel()(z0, ei0[0], ei0[1], z1, ei1[0], ei1[1], zrows)
    return p0.reshape(2, NP, H), p1.reshape(2, NP, H)


def _pad_edges(ei):
    return jnp.pad(ei, ((0, 0), (0, E2 - E)), constant_values=N)


def _pad_mlp(rs, l, W0, mlp_W1, mlp_b1, mlp_W2, mlp_b2, mlp_W3, mlp_b3, rank1):
    pads = HMP - HM
    W1s, b1s, W2s, b2s, W3s, b3s = [], [], [], [], [], []
    for r in rs:
        W1 = mlp_W1[l, r]
        if rank1:
            W1 = W0 @ W1
        W1s.append(jnp.pad(W1, ((0, 0), (0, pads))))
        b1s.append(jnp.pad(mlp_b1[l, r][None], ((0, 0), (0, pads))))
        W2s.append(jnp.pad(mlp_W2[l, r], ((0, pads), (0, pads))))
        b2s.append(jnp.pad(mlp_b2[l, r][None], ((0, 0), (0, pads))))
        W3s.append(jnp.pad(mlp_W3[l, r], ((0, pads), (0, 0))))
        b3s.append(mlp_b3[l, r][None])
    return (jnp.stack(W1s), jnp.stack(b1s), jnp.stack(W2s), jnp.stack(b2s),
            jnp.stack(W3s), jnp.stack(b3s))


def kernel(x_clause, x_variable, deg_clause, deg_variable, ei_cp, ei_cn,
           ei_rp, ei_rn, W0c, W0v, conv_ls_W, conv_ls_b, mlp_W1, mlp_b1,
           mlp_W2, mlp_b2, mlp_W3, mlp_b3, lins_c_W, lins_c_b, lins_v_W,
           lins_v_b):
    del conv_ls_W, conv_ls_b
    pad = NP - N
    xc = jnp.pad(x_clause, ((0, pad), (0, 0)))
    xv = jnp.pad(x_variable, ((0, pad), (0, 0)))
    degc = jnp.pad(deg_clause.reshape(N, 1), ((0, pad), (0, 0)))
    degv = jnp.pad(deg_variable.reshape(N, 1), ((0, pad), (0, 0)))
    e_cp, e_cn = _pad_edges(ei_cp), _pad_edges(ei_cn)
    e_rp, e_rn = _pad_edges(ei_rp), _pad_edges(ei_rn)
    zrows = jnp.zeros((RPT, H), F32)

    wc = _pad_mlp((0, 1), 0, W0c, mlp_W1, mlp_b1, mlp_W2, mlp_b2, mlp_W3,
                  mlp_b3, rank1=True)
    zc0, zc1 = _mlp_pair(xc, degc, *wc, rank1=True)
    wv = _pad_mlp((2, 3), 0, W0v, mlp_W1, mlp_b1, mlp_W2, mlp_b2, mlp_W3,
                  mlp_b3, rank1=True)
    zv0, zv1 = _mlp_pair(xv, degv, *wv, rank1=True)

    pcp, pcn = _conv_pair(zv0, e_rp, zv1, e_rn, zrows)
    pvp, pvn = _conv_pair(zc0, e_cp, zc1, e_cn, zrows)

    xc1 = _combine(pcp, pcn, degc, xc,
                   lins_c_W[0, :H], lins_c_W[0, H:2 * H],
                   W0c @ lins_c_W[0, 2 * H:], lins_c_b[0][None], rank1=True)
    xv1 = _combine(pvp, pvn, degv, xv,
                   lins_v_W[0, :H], lins_v_W[0, H:2 * H],
                   W0v @ lins_v_W[0, 2 * H:], lins_v_b[0][None], rank1=True)

    wc1 = _pad_mlp((0, 1), 1, None, mlp_W1, mlp_b1, mlp_W2, mlp_b2, mlp_W3,
                   mlp_b3, rank1=False)
    zq0, zq1 = _mlp_pair(xc1, degc, *wc1, rank1=False)
    qvp, qvn = _conv_pair(zq0, e_cp, zq1, e_cn, zrows)

    xv2 = _combine(qvp, qvn, degv, xv1,
                   lins_v_W[1, :H], lins_v_W[1, H:2 * H],
                   lins_v_W[1, 2 * H:], lins_v_b[1][None], rank1=False)
    return xv2[:N]


# speedup vs baseline: 3.2372x; 3.2372x over previous
import functools

import jax
import jax.numpy as jnp
from jax import lax
from jax.experimental import pallas as pl
from jax.experimental.pallas import tpu as pltpu
from jax.experimental.pallas import tpu_sc as plsc

H = 128
HM = 153
HMP = 160
N = 10000
E = 160000
F32 = jnp.float32

NCORES = 2
NSUB = 16
NW = NCORES * NSUB
CHUNK = 128
NJ = 40
E2 = NW * NJ * CHUNK
NCH = E2 // CHUNK
NB = 2
NG = NJ // NB
NP = 10240
RPT = NP // NSUB
BR = 1024


def _inv_sqrt(d):
    safe = jnp.where(d > 0, d, 1.0)
    return jnp.where(d > 0, lax.rsqrt(safe), 0.0)


def _mlp_pair_body(rank1, x_ref, deg_ref, W1_ref, b1_ref, W2_ref, b2_ref,
                   W3_ref, b3_ref, oa_ref, ob_ref):
    dsi = _inv_sqrt(deg_ref[...])
    x = x_ref[...]
    for k, out in ((0, oa_ref), (1, ob_ref)):
        if rank1:
            h = x * W1_ref[k] + b1_ref[k]
        else:
            h = jnp.dot(x, W1_ref[k], preferred_element_type=F32) + b1_ref[k]
        h = jnp.maximum(h, 0.0)
        h = jnp.maximum(jnp.dot(h, W2_ref[k], preferred_element_type=F32) + b2_ref[k], 0.0)
        h = jnp.maximum(jnp.dot(h, W3_ref[k], preferred_element_type=F32) + b3_ref[k], 0.0)
        out[...] = h * dsi


def _mlp_pair(x, deg, W1, b1, W2, b2, W3, b3, rank1):
    grid = (NP // BR,)
    kdim = 1 if rank1 else H
    full = lambda *s: pl.BlockSpec(s, lambda i: (0,) * len(s))
    return pl.pallas_call(
        functools.partial(_mlp_pair_body, rank1),
        grid=grid,
        in_specs=[
            pl.BlockSpec((BR, kdim), lambda i: (i, 0)),
            pl.BlockSpec((BR, 1), lambda i: (i, 0)),
            full(2, kdim, HMP), full(2, 1, HMP),
            full(2, HMP, HMP), full(2, 1, HMP),
            full(2, HMP, H), full(2, 1, H),
        ],
        out_specs=[pl.BlockSpec((BR, H), lambda i: (i, 0))] * 2,
        out_shape=[jax.ShapeDtypeStruct((NP, H), F32)] * 2,
    )(x, deg, W1, b1, W2, b2, W3, b3)


def _combine_body(rank1, pp_ref, pn_ref, deg_ref, prev_ref, Wa_ref, Wb_ref,
                  Wc_ref, b_ref, out_ref):
    dti = _inv_sqrt(deg_ref[...])
    a = (pp_ref[0] + pp_ref[1]) * dti
    c = (pn_ref[0] + pn_ref[1]) * dti
    acc = jnp.dot(a, Wa_ref[...], preferred_element_type=F32)
    acc += jnp.dot(c, Wb_ref[...], preferred_element_type=F32)
    if rank1:
        acc += prev_ref[...] * Wc_ref[...]
    else:
        acc += jnp.dot(prev_ref[...], Wc_ref[...], preferred_element_type=F32)
    out_ref[...] = acc + b_ref[...]


def _combine(pp, pn, deg, prev, Wa, Wb, Wc, b, rank1):
    grid = (NP // BR,)
    kdim = 1 if rank1 else H
    full = lambda *s: pl.BlockSpec(s, lambda i: (0,) * len(s))
    return pl.pallas_call(
        functools.partial(_combine_body, rank1),
        grid=grid,
        in_specs=[
            pl.BlockSpec((2, BR, H), lambda i: (0, i, 0)),
            pl.BlockSpec((2, BR, H), lambda i: (0, i, 0)),
            pl.BlockSpec((BR, 1), lambda i: (i, 0)),
            pl.BlockSpec((BR, kdim), lambda i: (i, 0)),
            full(H, H), full(H, H), full(kdim, H), full(1, H),
        ],
        out_specs=pl.BlockSpec((BR, H), lambda i: (i, 0)),
        out_shape=jax.ShapeDtypeStruct((NP, H), F32),
    )(pp, pn, deg, prev, Wa, Wb, Wc, b)


def _conv2_body(z0, s0, t0, z1, s1, t1, zrows, out0, out1,
                acc, sbuf, tbuf, rows, semg, sems):
    cid = lax.axis_index("c")
    sid = lax.axis_index("s")
    wid = sid * NCORES + cid
    base = sid * RPT
    for (z, s, t, out) in ((z0, s0, t0, out0), (z1, s1, t1, out1)):
        pltpu.sync_copy(s.at[wid], sbuf)
        pltpu.sync_copy(t.at[wid], tbuf)
        pltpu.sync_copy(zrows, acc.at[pl.ds(base, RPT)])
        plsc.subcore_barrier()

        pltpu.async_copy(z.at[sbuf.at[0]], rows.at[0], semg[0])

        @pl.loop(0, NJ // 2)
        def _(g):
            for b in (0, 1):
                j = g * 2 + b
                jn = jnp.minimum(j + 1, NJ - 1)
                pltpu.make_async_copy(z.at[sbuf.at[j]], rows.at[b],
                                      semg[b]).wait()
                pltpu.async_copy(z.at[sbuf.at[jn]], rows.at[1 - b],
                                 semg[1 - b])
                pltpu.sync_copy(rows.at[b], acc.at[tbuf.at[j]], add=True)

        pltpu.make_async_copy(z.at[sbuf.at[NJ - 1]], rows.at[0],
                              semg[0]).wait()

        plsc.subcore_barrier()
        pltpu.sync_copy(acc.at[pl.ds(base, RPT)],
                        out.at[pl.ds(cid * NP + base, RPT)])


@functools.cache
def _conv2_kernel():
    mesh = plsc.VectorSubcoreMesh(core_axis_name="c", subcore_axis_name="s")
    return pl.kernel(
        _conv2_body,
        mesh=mesh,
        out_type=[jax.ShapeDtypeStruct((NCORES * NP, H), F32)] * 2,
        scratch_types=[
            pltpu.VMEM_SHARED((NP, H), F32),
            pltpu.VMEM((NJ, CHUNK), jnp.int32),
            pltpu.VMEM((NJ, CHUNK), jnp.int32),
            pltpu.VMEM((NB, CHUNK, H), F32),
            [pltpu.SemaphoreType.DMA] * NB,
            [pltpu.SemaphoreType.DMA] * NB,
        ],
    )


def _conv_pair(z0, ei0, z1, ei1, zrows):
    p0, p1 = _conv2_kernel()(z0, ei0[0], ei0[1], z1, ei1[0], ei1[1], zrows)
    return p0.reshape(2, NP, H), p1.reshape(2, NP, H)


def _pad_edges(ei):
    r = jnp.arange(E2 - E, dtype=jnp.int32)
    src = jnp.concatenate([ei[0], N + r % (NP - N)])
    trg = jnp.concatenate([ei[1], (r * 67) % NP])
    return jnp.stack([src, trg]).reshape(2, NW, NJ, CHUNK)


def _pad_mlp(rs, l, W0, mlp_W1, mlp_b1, mlp_W2, mlp_b2, mlp_W3, mlp_b3, rank1):
    pads = HMP - HM
    W1s, b1s, W2s, b2s, W3s, b3s = [], [], [], [], [], []
    for r in rs:
        W1 = mlp_W1[l, r]
        if rank1:
            W1 = W0 @ W1
        W1s.append(jnp.pad(W1, ((0, 0), (0, pads))))
        b1s.append(jnp.pad(mlp_b1[l, r][None], ((0, 0), (0, pads))))
        W2s.append(jnp.pad(mlp_W2[l, r], ((0, pads), (0, pads))))
        b2s.append(jnp.pad(mlp_b2[l, r][None], ((0, 0), (0, pads))))
        W3s.append(jnp.pad(mlp_W3[l, r], ((0, pads), (0, 0))))
        b3s.append(mlp_b3[l, r][None])
    return (jnp.stack(W1s), jnp.stack(b1s), jnp.stack(W2s), jnp.stack(b2s),
            jnp.stack(W3s), jnp.stack(b3s))


def kernel(x_clause, x_variable, deg_clause, deg_variable, ei_cp, ei_cn,
           ei_rp, ei_rn, W0c, W0v, conv_ls_W, conv_ls_b, mlp_W1, mlp_b1,
           mlp_W2, mlp_b2, mlp_W3, mlp_b3, lins_c_W, lins_c_b, lins_v_W,
           lins_v_b):
    del conv_ls_W, conv_ls_b
    pad = NP - N
    xc = jnp.pad(x_clause, ((0, pad), (0, 0)))
    xv = jnp.pad(x_variable, ((0, pad), (0, 0)))
    degc = jnp.pad(deg_clause.reshape(N, 1), ((0, pad), (0, 0)))
    degv = jnp.pad(deg_variable.reshape(N, 1), ((0, pad), (0, 0)))
    e_cp, e_cn = _pad_edges(ei_cp), _pad_edges(ei_cn)
    e_rp, e_rn = _pad_edges(ei_rp), _pad_edges(ei_rn)
    zrows = jnp.zeros((RPT, H), F32)

    wc = _pad_mlp((0, 1), 0, W0c, mlp_W1, mlp_b1, mlp_W2, mlp_b2, mlp_W3,
                  mlp_b3, rank1=True)
    zc0, zc1 = _mlp_pair(xc, degc, *wc, rank1=True)
    wv = _pad_mlp((2, 3), 0, W0v, mlp_W1, mlp_b1, mlp_W2, mlp_b2, mlp_W3,
                  mlp_b3, rank1=True)
    zv0, zv1 = _mlp_pair(xv, degv, *wv, rank1=True)

    pcp, pcn = _conv_pair(zv0, e_rp, zv1, e_rn, zrows)
    pvp, pvn = _conv_pair(zc0, e_cp, zc1, e_cn, zrows)

    xc1 = _combine(pcp, pcn, degc, xc,
                   lins_c_W[0, :H], lins_c_W[0, H:2 * H],
                   W0c @ lins_c_W[0, 2 * H:], lins_c_b[0][None], rank1=True)
    xv1 = _combine(pvp, pvn, degv, xv,
                   lins_v_W[0, :H], lins_v_W[0, H:2 * H],
                   W0v @ lins_v_W[0, 2 * H:], lins_v_b[0][None], rank1=True)

    wc1 = _pad_mlp((0, 1), 1, None, mlp_W1, mlp_b1, mlp_W2, mlp_b2, mlp_W3,
                   mlp_b3, rank1=False)
    zq0, zq1 = _mlp_pair(xc1, degc, *wc1, rank1=False)
    qvp, qvn = _conv_pair(zq0, e_cp, zq1, e_cn, zrows)

    xv2 = _combine(qvp, qvn, degv, xv1,
                   lins_v_W[1, :H], lins_v_W[1, H:2 * H],
                   lins_v_W[1, 2 * H:], lins_v_b[1][None], rank1=False)
    return xv2[:N]


# speedup vs baseline: 3.5060x; 1.0831x over previous
import functools

import jax
import jax.numpy as jnp
import numpy as np
from jax import lax
from jax.experimental import pallas as pl
from jax.experimental.pallas import tpu as pltpu
from jax.experimental.pallas import tpu_sc as plsc

H = 128
HM = 153
N = 10000
E = 160000
F32 = jnp.float32

NCORES = 2
NSUB = 16
NW = NCORES * NSUB
CHUNK = 128
NJ = 40
NJC = NJ * CHUNK
E2 = NW * NJC
NP = 10240
RPT = NP // NSUB
BR = 1024

_R = np.arange(E2 - E, dtype=np.int32)
_DUMMY = np.stack([N + _R % (NP - N), (_R * 67) % NP]).astype(np.int32)


def _inv_sqrt(d):
    safe = jnp.where(d > 0, d, 1.0)
    return jnp.where(d > 0, lax.rsqrt(safe), 0.0)


def _mlp_pair_body(rank1, x_ref, deg_ref, W1_ref, b1_ref, W2_ref, b2_ref,
                   W3_ref, b3_ref, W0_ref, oa_ref, ob_ref):
    dsi = _inv_sqrt(deg_ref[...])
    x = x_ref[...]
    for k, out in ((0, oa_ref), (1, ob_ref)):
        if rank1:
            w1e = jnp.dot(W0_ref[...], W1_ref[k], preferred_element_type=F32)
            h = x * w1e + b1_ref[k]
        else:
            h = jnp.dot(x, W1_ref[k], preferred_element_type=F32) + b1_ref[k]
        h = jnp.maximum(h, 0.0)
        h = jnp.maximum(jnp.dot(h, W2_ref[k], preferred_element_type=F32) + b2_ref[k], 0.0)
        h = jnp.maximum(jnp.dot(h, W3_ref[k], preferred_element_type=F32) + b3_ref[k], 0.0)
        out[...] = h * dsi


def _mlp_pair(x, deg, W0, W1, b1, W2, b2, W3, b3, rank1):
    grid = (NP // BR,)
    full = lambda *s: pl.BlockSpec(s, lambda i: (0,) * len(s))
    return pl.pallas_call(
        functools.partial(_mlp_pair_body, rank1),
        grid=grid,
        in_specs=[
            pl.BlockSpec((BR, 1 if rank1 else H), lambda i: (i, 0)),
            pl.BlockSpec((BR, 1), lambda i: (i, 0)),
            full(2, H, HM), full(2, 1, HM),
            full(2, HM, HM), full(2, 1, HM),
            full(2, HM, H), full(2, 1, H),
            full(1, H),
        ],
        out_specs=[pl.BlockSpec((BR, H), lambda i: (i, 0))] * 2,
        out_shape=[jax.ShapeDtypeStruct((NP, H), F32)] * 2,
    )(x, deg, W1, b1, W2, b2, W3, b3, W0)


def _combine_body(rank1, pp_ref, pn_ref, deg_ref, prev_ref, w0_ref, W_ref,
                  b_ref, out_ref):
    dti = _inv_sqrt(deg_ref[...])
    a = (pp_ref[0] + pp_ref[1]) * dti
    c = (pn_ref[0] + pn_ref[1]) * dti
    acc = jnp.dot(a, W_ref[0], preferred_element_type=F32)
    acc += jnp.dot(c, W_ref[1], preferred_element_type=F32)
    if rank1:
        we = jnp.dot(w0_ref[...], W_ref[2], preferred_element_type=F32)
        acc += prev_ref[...] * we
    else:
        acc += jnp.dot(prev_ref[...], W_ref[2], preferred_element_type=F32)
    out_ref[...] = acc + b_ref[...]


def _combine(pp, pn, deg, prev, W3s, b, rank1, w0=None, rows=NP):
    grid = (10,)
    br = rows // 10
    full = lambda *s: pl.BlockSpec(s, lambda i: (0,) * len(s))
    if w0 is None:
        w0 = jnp.zeros((1, H), F32)
    return pl.pallas_call(
        functools.partial(_combine_body, rank1),
        grid=grid,
        in_specs=[
            pl.BlockSpec((2, br, H), lambda i: (0, i, 0)),
            pl.BlockSpec((2, br, H), lambda i: (0, i, 0)),
            pl.BlockSpec((br, 1), lambda i: (i, 0)),
            pl.BlockSpec((br, 1 if rank1 else H), lambda i: (i, 0)),
            full(1, H), full(3, H, H), full(1, H),
        ],
        out_specs=pl.BlockSpec((br, H), lambda i: (i, 0)),
        out_shape=jax.ShapeDtypeStruct((rows, H), F32),
    )(pp, pn, deg, prev, w0, W3s, b)


def _conv2_body(z0, s0, t0, z1, s1, t1, zrows, out0, out1,
                acc, sbuf, tbuf, rows, semg, semt):
    cid = lax.axis_index("c")
    sid = lax.axis_index("s")
    wid = sid * NCORES + cid
    base = sid * RPT
    ebase = wid * NJC
    for (z, s, t, out) in ((z0, s0, t0, out0), (z1, s1, t1, out1)):
        pltpu.sync_copy(s.at[pl.ds(ebase, NJC)], sbuf)
        pltpu.sync_copy(zrows, acc.at[pl.ds(base, RPT)])
        plsc.subcore_barrier()

        def gat(j, b):
            pltpu.async_copy(z.at[sbuf.at[pl.ds(j * CHUNK, CHUNK)]],
                             rows.at[b], semg[b])

        def tcp(j, b):
            pltpu.async_copy(t.at[pl.ds(ebase + j * CHUNK, CHUNK)],
                             tbuf.at[b], semt[b])

        gat(0, 0)
        tcp(0, 0)

        @pl.loop(0, NJ // 2)
        def _(g):
            for b in (0, 1):
                j = g * 2 + b
                jn = jnp.minimum(j + 1, NJ - 1)
                gat(jn, 1 - b)
                tcp(jn, 1 - b)
                pltpu.make_async_copy(z.at[sbuf.at[pl.ds(0, CHUNK)]],
                                      rows.at[b], semg[b]).wait()
                pltpu.make_async_copy(t.at[pl.ds(0, CHUNK)],
                                      tbuf.at[b], semt[b]).wait()
                pltpu.sync_copy(rows.at[b], acc.at[tbuf.at[b]], add=True)

        pltpu.make_async_copy(z.at[sbuf.at[pl.ds(0, CHUNK)]],
                              rows.at[0], semg[0]).wait()
        pltpu.make_async_copy(t.at[pl.ds(0, CHUNK)], tbuf.at[0],
                              semt[0]).wait()

        plsc.subcore_barrier()
        pltpu.sync_copy(acc.at[pl.ds(base, RPT)],
                        out.at[pl.ds(cid * NP + base, RPT)])


@functools.cache
def _conv2_kernel():
    mesh = plsc.VectorSubcoreMesh(core_axis_name="c", subcore_axis_name="s")
    return pl.kernel(
        _conv2_body,
        mesh=mesh,
        out_type=[jax.ShapeDtypeStruct((NCORES * NP, H), F32)] * 2,
        scratch_types=[
            pltpu.VMEM_SHARED((NP, H), F32),
            pltpu.VMEM((NJC,), jnp.int32),
            pltpu.VMEM((2, CHUNK), jnp.int32),
            pltpu.VMEM((2, CHUNK, H), F32),
            [pltpu.SemaphoreType.DMA] * 2,
            [pltpu.SemaphoreType.DMA] * 2,
        ],
    )


def _conv_pair(z0, ei0, z1, ei1, zrows):
    p0, p1 = _conv2_kernel()(z0, ei0[0], ei0[1], z1, ei1[0], ei1[1], zrows)
    return p0.reshape(2, NP, H), p1.reshape(2, NP, H)


def _pad_edges(ei):
    return jnp.concatenate([ei, jnp.asarray(_DUMMY)], axis=1)


def kernel(x_clause, x_variable, deg_clause, deg_variable, ei_cp, ei_cn,
           ei_rp, ei_rn, W0c, W0v, conv_ls_W, conv_ls_b, mlp_W1, mlp_b1,
           mlp_W2, mlp_b2, mlp_W3, mlp_b3, lins_c_W, lins_c_b, lins_v_W,
           lins_v_b):
    del conv_ls_W, conv_ls_b
    pad = NP - N
    xc = jnp.pad(x_clause, ((0, pad), (0, 0)))
    xv = jnp.pad(x_variable, ((0, pad), (0, 0)))
    degc = jnp.pad(deg_clause.reshape(N, 1), ((0, pad), (0, 0)))
    degv = jnp.pad(deg_variable.reshape(N, 1), ((0, pad), (0, 0)))
    e_cp, e_cn = _pad_edges(ei_cp), _pad_edges(ei_cn)
    e_rp, e_rn = _pad_edges(ei_rp), _pad_edges(ei_rn)
    zrows = jnp.zeros((RPT, H), F32)

    def mw(l, r0):
        return (mlp_W1[l, r0:r0 + 2], mlp_b1[l, r0:r0 + 2, None],
                mlp_W2[l, r0:r0 + 2], mlp_b2[l, r0:r0 + 2, None],
                mlp_W3[l, r0:r0 + 2], mlp_b3[l, r0:r0 + 2, None])

    zv0, zv1 = _mlp_pair(xv, degv, W0v, *mw(0, 2), rank1=True)
    zc0, zc1 = _mlp_pair(xc, degc, W0c, *mw(0, 0), rank1=True)

    pcp, pcn = _conv_pair(zv0, e_rp, zv1, e_rn, zrows)
    pvp, pvn = _conv_pair(zc0, e_cp, zc1, e_cn, zrows)

    xc1 = _combine(pcp, pcn, degc, xc, lins_c_W[0].reshape(3, H, H),
                   lins_c_b[0][None], rank1=True, w0=W0c)
    xv1 = _combine(pvp, pvn, degv, xv, lins_v_W[0].reshape(3, H, H),
                   lins_v_b[0][None], rank1=True, w0=W0v)

    zq0, zq1 = _mlp_pair(xc1, degc, W0c, *mw(1, 0), rank1=False)
    qvp, qvn = _conv_pair(zq0, e_cp, zq1, e_cn, zrows)

    return _combine(qvp, qvn, degv, xv1, lins_v_W[1].reshape(3, H, H),
                    lins_v_b[1][None], rank1=False, rows=N)
